# baseline (speedup 1.0000x reference)
import functools

import jax
import jax.numpy as jnp
from jax import lax
from jax.experimental import pallas as pl
from jax.experimental.pallas import tpu as pltpu
from jax.experimental.pallas import tpu_sc as plsc

N_REL = 8
D = 32
NC = 2
NS = 16
L = 16
NW = NC * NS


def _rsqrt(x):
    i = plsc.bitcast(x, jnp.int32)
    y = plsc.bitcast(jnp.int32(0x5F3759DF) - (i >> 1), jnp.float32)
    for _ in range(3):
        y = y * (1.5 - 0.5 * x * y * y)
    return y


def _make_rel_kernel(B, b_per_w, n_chunks, r, first, last):
    mesh = plsc.VectorSubcoreMesh(core_axis_name="c", subcore_axis_name="s")
    n_in = 5 if first else 6

    scratch = [
        pltpu.VMEM((b_per_w,), jnp.int32),
        pltpu.VMEM((b_per_w,), jnp.int32),
        pltpu.VMEM((b_per_w, D), jnp.float32),
        pltpu.VMEM((b_per_w, D), jnp.float32),
        pltpu.VMEM((b_per_w,), jnp.float32),
        pltpu.VMEM((L,), jnp.float32),
        pltpu.SemaphoreType.DMA,
        pltpu.SemaphoreType.DMA,
    ]

    def body(*refs):
        if first:
            u_hbm, v_hbm, ut_hbm, it_hbm, w_hbm, out_hbm = refs[:6]
            acc_hbm = None
        else:
            u_hbm, v_hbm, ut_hbm, it_hbm, w_hbm, acc_hbm, out_hbm = refs[:7]
        iu, iv, ru, rv, acc, wv, s_u, s_v = refs[n_in + 1:]

        wid = lax.axis_index("s") * NC + lax.axis_index("c")
        base = wid * b_per_w
        pltpu.sync_copy(u_hbm.at[pl.ds(base, b_per_w)], iu)
        pltpu.sync_copy(v_hbm.at[pl.ds(base, b_per_w)], iv)
        pltpu.sync_copy(w_hbm, wv)
        wr = wv[pl.ds(0, L)][r]

        cu = pltpu.async_copy(ut_hbm.at[iu], ru, s_u)
        cv = pltpu.async_copy(it_hbm.at[iv], rv, s_v)

        if not first:
            pltpu.sync_copy(acc_hbm.at[pl.ds(base, b_per_w)], acc)

        cu.wait()
        cv.wait()

        def comp_body(c, carry):
            row_ids = c * L + lax.iota(jnp.int32, L)
            duv = jnp.zeros((L,), jnp.float32)
            su_a = jnp.zeros((L,), jnp.float32)
            sv_a = jnp.zeros((L,), jnp.float32)
            for d in range(D):
                col = jnp.full((L,), d, jnp.int32)
                xu = plsc.load_gather(ru, [row_ids, col])
                xv = plsc.load_gather(rv, [row_ids, col])
                duv = duv + xu * xv
                su_a = su_a + xu * xu
                sv_a = sv_a + xv * xv
            denom = jnp.maximum(su_a, 1.0) * jnp.maximum(sv_a, 1.0)
            val = duv * _rsqrt(denom) * wr
            s = pl.ds(c * L, L)
            if first:
                x = val
            else:
                x = acc[s] + val
            if last:
                x = 1.0 / (1.0 + jnp.exp(-x))
            acc[s] = x
            return carry
        lax.fori_loop(0, n_chunks, comp_body, 0)

        pltpu.sync_copy(acc, out_hbm.at[pl.ds(base, b_per_w)])

    return pl.kernel(
        body,
        mesh=mesh,
        out_type=jax.ShapeDtypeStruct((B,), jnp.float32),
        compiler_params=pltpu.CompilerParams(
            needs_layout_passes=False, use_tc_tiling_on_sc=False),
        scratch_types=scratch,
    )


def kernel(u, v, user_tables, item_tables, r_weight):
    B = u.shape[0]
    b_per_w = B // NW
    n_chunks = b_per_w // L
    w = jnp.pad(r_weight.reshape(-1), (0, L - N_REL))

    acc = None
    for r in range(N_REL):
        ut_r = user_tables[r]
        it_r = item_tables[r]
        k = _make_rel_kernel(B, b_per_w, n_chunks, r,
                             first=(r == 0), last=(r == N_REL - 1))
        if r == 0:
            acc = k(u, v, ut_r, it_r, w)
        else:
            acc = k(u, v, ut_r, it_r, w, acc)
    return acc


# speedup vs baseline: 2.0163x; 2.0163x over previous
import functools

import jax
import jax.numpy as jnp
from jax import lax
from jax.experimental import pallas as pl
from jax.experimental.pallas import tpu as pltpu
from jax.experimental.pallas import tpu_sc as plsc

N_REL = 8
D = 32
G = 2
NC = 2
NS = 16
L = 16
NW = NC * NS


def _rsqrt(x):
    i = plsc.bitcast(x, jnp.int32)
    y = plsc.bitcast(jnp.int32(0x5F3759DF) - (i >> 1), jnp.float32)
    for _ in range(3):
        y = y * (1.5 - 0.5 * x * y * y)
    return y


def _make_group_kernel(B, b_per_w, n_chunks, r0, first, last):
    mesh = plsc.VectorSubcoreMesh(core_axis_name="c", subcore_axis_name="s")
    n_in = 3 + 2 * G if first else 4 + 2 * G

    scratch = [
        pltpu.VMEM((b_per_w,), jnp.int32),
        pltpu.VMEM((b_per_w,), jnp.int32),
        pltpu.VMEM((G, D, b_per_w), jnp.float32),
        pltpu.VMEM((G, D, b_per_w), jnp.float32),
        pltpu.VMEM((b_per_w,), jnp.float32),
        pltpu.VMEM((L,), jnp.float32),
    ] + [pltpu.SemaphoreType.DMA] * (2 * G)

    def body(*refs):
        tabs = refs[2:2 + 2 * G]
        if first:
            u_hbm, v_hbm = refs[:2]
            w_hbm, out_hbm = refs[2 + 2 * G:4 + 2 * G]
            acc_hbm = None
        else:
            u_hbm, v_hbm = refs[:2]
            w_hbm, acc_hbm, out_hbm = refs[2 + 2 * G:5 + 2 * G]
        iu, iv, eu, ev, acc, wv = refs[n_in + 1:n_in + 7]
        sems = refs[n_in + 7:]

        wid = lax.axis_index("s") * NC + lax.axis_index("c")
        base = wid * b_per_w
        pltpu.sync_copy(u_hbm.at[pl.ds(base, b_per_w)], iu)
        pltpu.sync_copy(v_hbm.at[pl.ds(base, b_per_w)], iv)

        def fire(g):
            ut_hbm, it_hbm = tabs[2 * g], tabs[2 * g + 1]

            def fire_body(d, carry):
                pltpu.async_copy(ut_hbm.at[d].at[iu], eu.at[g, d],
                                 sems[2 * g])
                pltpu.async_copy(it_hbm.at[d].at[iv], ev.at[g, d],
                                 sems[2 * g + 1])
                return carry
            lax.fori_loop(0, D, fire_body, 0)

        def drain(g):
            ut_hbm, it_hbm = tabs[2 * g], tabs[2 * g + 1]

            def drain_body(d, carry):
                pltpu.make_async_copy(
                    ut_hbm.at[d].at[iu], eu.at[g, d], sems[2 * g]).wait()
                pltpu.make_async_copy(
                    it_hbm.at[d].at[iv], ev.at[g, d], sems[2 * g + 1]).wait()
                return carry
            lax.fori_loop(0, D, drain_body, 0)

        for g in range(G):
            fire(g)

        pltpu.sync_copy(w_hbm, wv)
        wall = wv[pl.ds(0, L)]
        if not first:
            pltpu.sync_copy(acc_hbm.at[pl.ds(base, b_per_w)], acc)

        for g in range(G):
            drain(g)
            wr = wall[r0 + g]
            g_first = first and g == 0
            g_last = last and g == G - 1

            def comp_body(c, carry, _g=g, _first=g_first, _last=g_last,
                          _wr=wr):
                s = pl.ds(c * L, L)
                duv = jnp.zeros((L,), jnp.float32)
                su_a = jnp.zeros((L,), jnp.float32)
                sv_a = jnp.zeros((L,), jnp.float32)
                for d in range(D):
                    xu = eu[_g, d, s]
                    xv = ev[_g, d, s]
                    duv = duv + xu * xv
                    su_a = su_a + xu * xu
                    sv_a = sv_a + xv * xv
                denom = jnp.maximum(su_a, 1.0) * jnp.maximum(sv_a, 1.0)
                val = duv * _rsqrt(denom) * _wr
                if _first:
                    x = val
                else:
                    x = acc[s] + val
                if _last:
                    x = 1.0 / (1.0 + jnp.exp(-x))
                acc[s] = x
                return carry
            lax.fori_loop(0, n_chunks, comp_body, 0)

        pltpu.sync_copy(acc, out_hbm.at[pl.ds(base, b_per_w)])

    return pl.kernel(
        body,
        mesh=mesh,
        out_type=jax.ShapeDtypeStruct((B,), jnp.float32),
        compiler_params=pltpu.CompilerParams(
            needs_layout_passes=False, use_tc_tiling_on_sc=False),
        scratch_types=scratch,
    )


def kernel(u, v, user_tables, item_tables, r_weight):
    B = u.shape[0]
    b_per_w = B // NW
    n_chunks = b_per_w // L
    w = jnp.pad(r_weight.reshape(-1), (0, L - N_REL))

    acc = None
    for r0 in range(0, N_REL, G):
        tabs = []
        for g in range(G):
            tabs.append(user_tables[r0 + g].transpose(1, 0))
            tabs.append(item_tables[r0 + g].transpose(1, 0))
        k = _make_group_kernel(B, b_per_w, n_chunks, r0,
                               first=(r0 == 0), last=(r0 + G == N_REL))
        if r0 == 0:
            acc = k(u, v, *tabs, w)
        else:
            acc = k(u, v, *tabs, w, acc)
    return acc


# speedup vs baseline: 2.0532x; 1.0183x over previous
import functools

import jax
import jax.numpy as jnp
from jax import lax
from jax.experimental import pallas as pl
from jax.experimental.pallas import tpu as pltpu
from jax.experimental.pallas import tpu_sc as plsc

N_REL = 8
D = 32
NC = 2
NS = 16
L = 16
NW = NC * NS


def _rsqrt(x):
    i = plsc.bitcast(x, jnp.int32)
    y = plsc.bitcast(jnp.int32(0x5F3759DF) - (i >> 1), jnp.float32)
    for _ in range(3):
        y = y * (1.5 - 0.5 * x * y * y)
    return y


def _make_rel_kernel(B, b_per_w, n_chunks, r, first, last):
    mesh = plsc.VectorSubcoreMesh(core_axis_name="c", subcore_axis_name="s")
    n_in = 5 if first else 6

    scratch = [
        pltpu.VMEM((b_per_w,), jnp.int32),
        pltpu.VMEM((b_per_w,), jnp.int32),
        pltpu.VMEM((D, b_per_w), jnp.float32),
        pltpu.VMEM((D, b_per_w), jnp.float32),
        pltpu.VMEM((b_per_w,), jnp.float32),
        pltpu.VMEM((L,), jnp.float32),
        pltpu.SemaphoreType.DMA,
        pltpu.SemaphoreType.DMA,
    ]

    def body(*refs):
        if first:
            u_hbm, v_hbm, ut_hbm, it_hbm, w_hbm, out_hbm = refs[:6]
            acc_hbm = None
        else:
            u_hbm, v_hbm, ut_hbm, it_hbm, w_hbm, acc_hbm, out_hbm = refs[:7]
        iu, iv, eu, ev, acc, wv, s_u, s_v = refs[n_in + 1:]

        wid = lax.axis_index("s") * NC + lax.axis_index("c")
        base = wid * b_per_w
        pltpu.sync_copy(u_hbm.at[pl.ds(base, b_per_w)], iu)
        pltpu.sync_copy(v_hbm.at[pl.ds(base, b_per_w)], iv)
        pltpu.sync_copy(w_hbm, wv)
        wr = wv[pl.ds(0, L)][r]

        def fire_body(d, carry):
            pltpu.async_copy(ut_hbm.at[d].at[iu], eu.at[d], s_u)
            pltpu.async_copy(it_hbm.at[d].at[iv], ev.at[d], s_v)
            return carry
        lax.fori_loop(0, D, fire_body, 0)

        if not first:
            pltpu.sync_copy(acc_hbm.at[pl.ds(base, b_per_w)], acc)

        def drain_body(d, carry):
            pltpu.make_async_copy(ut_hbm.at[d].at[iu], eu.at[d], s_u).wait()
            pltpu.make_async_copy(it_hbm.at[d].at[iv], ev.at[d], s_v).wait()
            return carry
        lax.fori_loop(0, D, drain_body, 0)

        def comp_body(c, carry):
            s = pl.ds(c * L, L)
            duv = jnp.zeros((L,), jnp.float32)
            su_a = jnp.zeros((L,), jnp.float32)
            sv_a = jnp.zeros((L,), jnp.float32)
            for d in range(D):
                xu = eu[d, s]
                xv = ev[d, s]
                duv = duv + xu * xv
                su_a = su_a + xu * xu
                sv_a = sv_a + xv * xv
            denom = jnp.maximum(su_a, 1.0) * jnp.maximum(sv_a, 1.0)
            val = duv * _rsqrt(denom) * wr
            if first:
                x = val
            else:
                x = acc[s] + val
            if last:
                x = 1.0 / (1.0 + jnp.exp(-x))
            acc[s] = x
            return carry
        lax.fori_loop(0, n_chunks, comp_body, 0)

        pltpu.sync_copy(acc, out_hbm.at[pl.ds(base, b_per_w)])

    return pl.kernel(
        body,
        mesh=mesh,
        out_type=jax.ShapeDtypeStruct((B,), jnp.float32),
        compiler_params=pltpu.CompilerParams(
            needs_layout_passes=False, use_tc_tiling_on_sc=False),
        scratch_types=scratch,
    )


def kernel(u, v, user_tables, item_tables, r_weight):
    B = u.shape[0]
    b_per_w = B // NW
    n_chunks = b_per_w // L
    w = jnp.pad(r_weight.reshape(-1), (0, L - N_REL))

    acc = None
    for r in range(N_REL):
        ri = jnp.int32(r)
        ut_r = lax.dynamic_index_in_dim(
            user_tables, ri, 0, keepdims=False).transpose(1, 0)
        it_r = lax.dynamic_index_in_dim(
            item_tables, ri, 0, keepdims=False).transpose(1, 0)
        k = _make_rel_kernel(B, b_per_w, n_chunks, r,
                             first=(r == 0), last=(r == N_REL - 1))
        if r == 0:
            acc = k(u, v, ut_r, it_r, w)
        else:
            acc = k(u, v, ut_r, it_r, w, acc)
    return acc
